# Initial kernel scaffold; baseline (speedup 1.0000x reference)
import jax
import jax.numpy as jnp
from jax.experimental import pallas as pl


def kernel(x, edge_index, edge_attr, topic_signal, q_emb, non_text_emb, W1, b1, W2, b2):
    raise NotImplementedError("write your pallas kernel here")


# speedup vs baseline: 7.2493x; 7.2493x over previous
import functools

import jax
import jax.numpy as jnp
from jax import lax
from jax.experimental import pallas as pl
from jax.experimental.pallas import tpu as pltpu
from jax.experimental.pallas import tpu_sc as plsc

N = 10000
E = 320000
D = 128

NS = 16
SL = 632
NPAD = NS * SL
DUMMY = NPAD - 8
EPT = E // NS
NCH = (EPT + 127) // 128
EPAD = NS * NCH * 128

EW = E // 32
GCH = 200
GNC = EW // GCH


def _pe_conv_sc(topic_flat, gidx, sidx):
  mesh = plsc.VectorSubcoreMesh(core_axis_name="c", subcore_axis_name="s")

  @functools.partial(
      pl.kernel,
      out_type=[
          jax.ShapeDtypeStruct((4 * 2 * NPAD,), jnp.float32),
          jax.ShapeDtypeStruct((2, NS, 2 * NPAD), jnp.float32),
          jax.ShapeDtypeStruct((2, NS, NPAD), jnp.float32),
      ],
      mesh=mesh,
      compiler_params=pltpu.CompilerParams(needs_layout_passes=False, use_tc_tiling_on_sc=False),
      scratch_types=[
          pltpu.VMEM((2 * NPAD,), jnp.float32),
          pltpu.VMEM((NCH, 128), jnp.int32),
          pltpu.VMEM((NCH, 128), jnp.int32),
          pltpu.VMEM((2 * NPAD,), jnp.float32),
          pltpu.VMEM((NPAD,), jnp.float32),
          pltpu.VMEM_SHARED((2, 2 * NPAD), jnp.float32),
          pltpu.VMEM((1280,), jnp.float32),
          pltpu.VMEM((1280,), jnp.float32),
          pltpu.VMEM((640,), jnp.float32),
          pltpu.VMEM((1280,), jnp.float32),
      ],
  )
  def k(topic_hbm, gidx_hbm, sidx_hbm, out_hbm, parts_s, parts_c,
        tab, gv, sv, sums, cnts, h1sh, cb, tmp, ci, ob):
    c = lax.axis_index("c")
    s = lax.axis_index("s")
    n0 = s * SL
    iota = lax.iota(jnp.int32, 16)
    zeros16 = jnp.zeros((16,), jnp.float32)
    ones16 = jnp.ones((16,), jnp.float32)

    pltpu.sync_copy(topic_hbm, tab)
    pltpu.sync_copy(gidx_hbm.at[c, s], gv)
    pltpu.sync_copy(sidx_hbm.at[c, s], sv)

    def zero_buf(buf, nvregs):
      def zb(i, carry):
        buf[pl.ds(i * 16, 16)] = zeros16
        return carry
      lax.fori_loop(0, nvregs, zb, 0)

    def accum_round(with_counts):
      def chunk(j, carry):
        for i in range(8):
          g16 = gv[j, pl.ds(i * 16, 16)]
          d16 = sv[j, pl.ds(i * 16, 16)]
          m0 = plsc.load_gather(tab, [g16 * 2])
          m1 = plsc.load_gather(tab, [g16 * 2 + 1])
          d2 = d16 * 2
          plsc.addupdate_scatter(sums, [d2], m0)
          plsc.addupdate_scatter(sums, [d2 + 1], m1)
          if with_counts:
            plsc.addupdate_scatter(cnts, [d16], ones16)
        return carry
      lax.fori_loop(0, NCH, chunk, 0)

    def vadd_into(dst, nvregs):
      def body(i, carry):
        sl = pl.ds(i * 16, 16)
        dst[sl] = dst[sl] + tmp[sl]
        return carry
      lax.fori_loop(0, nvregs, body, 0)

    def combine(first_round):
      pltpu.sync_copy(sums, parts_s.at[c, s])
      if first_round:
        pltpu.sync_copy(cnts, parts_c.at[c, s])
      plsc.subcore_barrier()
      zero_buf(cb, 80)
      for t in range(NS):
        pltpu.sync_copy(parts_s.at[c, t, pl.ds(2 * n0, 2 * SL)],
                        tmp.at[pl.ds(0, 2 * SL)])
        vadd_into(cb, 79)
      if first_round:
        zero_buf(ci, 40)
        for t in range(NS):
          pltpu.sync_copy(parts_c.at[c, t, pl.ds(n0, SL)],
                          tmp.at[pl.ds(0, SL)])
          def addci(i, carry):
            sl = pl.ds(i * 16, 16)
            ci[sl] = ci[sl] + tmp[sl]
            return carry
          lax.fori_loop(0, 40, addci, 0)
      def nodes(k_, carry):
        ln = k_ * 16 + iota
        s0 = plsc.load_gather(cb, [ln * 2])
        s1 = plsc.load_gather(cb, [ln * 2 + 1])
        cn = ci[pl.ds(k_ * 16, 16)]
        inv = 1.0 / jnp.maximum(cn, 1.0)
        plsc.store_scatter(ob, [ln * 2], s0 * inv)
        plsc.store_scatter(ob, [ln * 2 + 1], s1 * inv)
        return carry
      lax.fori_loop(0, 40, nodes, 0)

    zero_buf(sums, 2 * NPAD // 16)
    zero_buf(cnts, NPAD // 16)
    accum_round(True)
    combine(True)
    pltpu.sync_copy(ob.at[pl.ds(0, 2 * SL)],
                    out_hbm.at[pl.ds(c * 2 * (2 * NPAD) + 2 * n0, 2 * SL)])
    pltpu.sync_copy(ob.at[pl.ds(0, 2 * SL)], h1sh.at[c, pl.ds(2 * n0, 2 * SL)])
    plsc.subcore_barrier()

    pltpu.sync_copy(h1sh.at[c], tab)
    zero_buf(sums, 2 * NPAD // 16)
    accum_round(False)
    combine(False)
    pltpu.sync_copy(ob.at[pl.ds(0, 2 * SL)],
                    out_hbm.at[pl.ds((c * 2 + 1) * (2 * NPAD) + 2 * n0, 2 * SL)])

  return k(topic_flat, gidx, sidx)[0]


def _edge_gather_sc(p_tab, q_tab, src, dst):
  mesh = plsc.VectorSubcoreMesh(core_axis_name="c", subcore_axis_name="s")

  @functools.partial(
      pl.kernel,
      out_type=jax.ShapeDtypeStruct((E, D), jnp.float32),
      mesh=mesh,
      compiler_params=pltpu.CompilerParams(needs_layout_passes=False, use_tc_tiling_on_sc=False),
      scratch_types=[
          pltpu.VMEM((GCH,), jnp.int32),
          pltpu.VMEM((GCH,), jnp.int32),
          pltpu.VMEM((GCH, D), jnp.float32),
          pltpu.VMEM((GCH, D), jnp.float32),
          pltpu.SemaphoreType.DMA,
          pltpu.SemaphoreType.DMA,
      ],
  )
  def k(p_hbm, q_hbm, src_hbm, dst_hbm, g_hbm, si, di, rp, rq, sem1, sem2):
    c = lax.axis_index("c")
    s = lax.axis_index("s")
    wid = s * 2 + c

    def chunk(j, carry):
      base = wid * EW + j * GCH
      pltpu.sync_copy(src_hbm.at[pl.ds(base, GCH)], si)
      pltpu.sync_copy(dst_hbm.at[pl.ds(base, GCH)], di)
      cp1 = pltpu.async_copy(p_hbm.at[si], rp, sem1)
      cp2 = pltpu.async_copy(q_hbm.at[di], rq, sem2)
      cp1.wait()
      cp2.wait()

      def row(r, carry2):
        for l in range(D // 16):
          sl = pl.ds(l * 16, 16)
          rp[r, sl] = rp[r, sl] + rq[r, sl]
        return carry2
      lax.fori_loop(0, GCH, row, 0)
      pltpu.sync_copy(rp, g_hbm.at[pl.ds(base, GCH)])
      return carry
    lax.fori_loop(0, GNC, chunk, 0)

  return k(p_tab, q_tab, src, dst)


def _node_mm_kernel(x_ref, e_ref, nte_ref, wsx_ref, wse_ref, wdx_ref, wde_ref,
                    p_ref, q_ref):
  xb = x_ref[...]
  mask = jnp.all(xb == 0.0, axis=1, keepdims=True)
  xm = jnp.where(mask, nte_ref[...], xb)
  ex = e_ref[...]
  p_ref[...] = (jnp.dot(xm, wsx_ref[...], preferred_element_type=jnp.float32)
                + jnp.dot(ex, wse_ref[...], preferred_element_type=jnp.float32))
  q_ref[...] = (jnp.dot(xm, wdx_ref[...], preferred_element_type=jnp.float32)
                + jnp.dot(ex, wde_ref[...], preferred_element_type=jnp.float32))


def _node_mm_tc(x, extras16, nte, wsx, wse, wdx, wde):
  bn = 1000
  grid = (N // bn,)
  return pl.pallas_call(
      _node_mm_kernel,
      grid=grid,
      in_specs=[
          pl.BlockSpec((bn, D), lambda i: (i, 0)),
          pl.BlockSpec((bn, 16), lambda i: (i, 0)),
          pl.BlockSpec((1, D), lambda i: (0, 0)),
          pl.BlockSpec((D, D), lambda i: (0, 0)),
          pl.BlockSpec((16, D), lambda i: (0, 0)),
          pl.BlockSpec((D, D), lambda i: (0, 0)),
          pl.BlockSpec((16, D), lambda i: (0, 0)),
      ],
      out_specs=[
          pl.BlockSpec((bn, D), lambda i: (i, 0)),
          pl.BlockSpec((bn, D), lambda i: (i, 0)),
      ],
      out_shape=[
          jax.ShapeDtypeStruct((N, D), jnp.float32),
          jax.ShapeDtypeStruct((N, D), jnp.float32),
      ],
  )(x, extras16, nte, wsx, wse, wdx, wde)


def _edge_mlp_kernel(q_ref, ea_ref, g_ref, w1q_ref, w1e_ref, b1_ref, w2_ref,
                     b2_ref, out_ref):
  h = jnp.dot(q_ref[...], w1q_ref[...], preferred_element_type=jnp.float32)
  h = h + jnp.dot(ea_ref[...], w1e_ref[...], preferred_element_type=jnp.float32)
  h = h + g_ref[...] + b1_ref[...]
  h = jnp.maximum(h, 0.0)
  out_ref[...] = jnp.dot(h, w2_ref[...], preferred_element_type=jnp.float32) + b2_ref[0]


def _edge_mlp_tc(q_emb, edge_attr, g, w1q, w1ea, b1r, w2, b2):
  be = 512
  grid = (E // be,)
  return pl.pallas_call(
      _edge_mlp_kernel,
      grid=grid,
      in_specs=[
          pl.BlockSpec((be, D), lambda i: (i, 0)),
          pl.BlockSpec((be, D), lambda i: (i, 0)),
          pl.BlockSpec((be, D), lambda i: (i, 0)),
          pl.BlockSpec((D, D), lambda i: (0, 0)),
          pl.BlockSpec((D, D), lambda i: (0, 0)),
          pl.BlockSpec((1, D), lambda i: (0, 0)),
          pl.BlockSpec((D, 1), lambda i: (0, 0)),
          pl.BlockSpec(memory_space=pltpu.SMEM),
      ],
      out_specs=pl.BlockSpec((be, 1), lambda i: (i, 0)),
      out_shape=jax.ShapeDtypeStruct((E, 1), jnp.float32),
  )(q_emb, edge_attr, g, w1q, w1ea, b1r, w2, b2)


def kernel(x, edge_index, edge_attr, topic_signal, q_emb, non_text_emb,
           W1, b1, W2, b2):
  src = edge_index[0]
  dst = edge_index[1]

  pad = jnp.full((EPAD - E,), DUMMY, jnp.int32)
  srcp = jnp.concatenate([src, pad]).reshape(NS, NCH, 128)
  dstp = jnp.concatenate([dst, pad]).reshape(NS, NCH, 128)
  gidx = jnp.stack([srcp, dstp])
  sidx = jnp.stack([dstp, srcp])
  topic_flat = jnp.pad(topic_signal.reshape(-1), (0, 2 * NPAD - 2 * N))
  pe = _pe_conv_sc(topic_flat, gidx, sidx).reshape(4, 2 * NPAD)
  f1 = pe[0, :2 * N].reshape(N, 2)
  f2 = pe[1, :2 * N].reshape(N, 2)
  r1 = pe[2, :2 * N].reshape(N, 2)
  r2 = pe[3, :2 * N].reshape(N, 2)

  extras16 = jnp.concatenate(
      [topic_signal, f1, f2, r1, r2, jnp.zeros((N, 6), jnp.float32)], axis=1)
  zpad6 = jnp.zeros((6, D), jnp.float32)
  wsx = W1[128:256]
  wse = jnp.concatenate([W1[256:266], zpad6], axis=0)
  wdx = W1[394:522]
  wde = jnp.concatenate([W1[522:532], zpad6], axis=0)
  p_tab, q_tab = _node_mm_tc(x, extras16, non_text_emb, wsx, wse, wdx, wde)

  g = _edge_gather_sc(p_tab, q_tab, src, dst)

  out = _edge_mlp_tc(q_emb, edge_attr, g, W1[0:128], W1[266:394],
                     b1.reshape(1, D), W2, b2)
  return out[:, 0]


# speedup vs baseline: 8.1828x; 1.1288x over previous
import functools

import jax
import jax.numpy as jnp
from jax import lax
from jax.experimental import pallas as pl
from jax.experimental.pallas import tpu as pltpu
from jax.experimental.pallas import tpu_sc as plsc

N = 10000
E = 320000
D = 128

NS = 16
SL = 632
NPAD = NS * SL
DUMMY = NPAD - 8
EPT = E // NS
NCH = (EPT + 127) // 128
EPAD = NS * NCH * 128

EW = E // 32
GCH = 200
GNC = EW // GCH


def _pe_conv_sc(topic_flat, gidx, sidx):
  mesh = plsc.VectorSubcoreMesh(core_axis_name="c", subcore_axis_name="s")

  @functools.partial(
      pl.kernel,
      out_type=[
          jax.ShapeDtypeStruct((4 * 2 * NPAD,), jnp.float32),
          jax.ShapeDtypeStruct((2, NS, 2 * NPAD), jnp.float32),
          jax.ShapeDtypeStruct((2, NS, NPAD), jnp.float32),
      ],
      mesh=mesh,
      compiler_params=pltpu.CompilerParams(needs_layout_passes=False, use_tc_tiling_on_sc=False),
      scratch_types=[
          pltpu.VMEM((2 * NPAD,), jnp.float32),
          pltpu.VMEM((NCH, 128), jnp.int32),
          pltpu.VMEM((NCH, 128), jnp.int32),
          pltpu.VMEM((2 * NPAD,), jnp.float32),
          pltpu.VMEM((NPAD,), jnp.float32),
          pltpu.VMEM_SHARED((2, 2 * NPAD), jnp.float32),
          pltpu.VMEM((1280,), jnp.float32),
          pltpu.VMEM((1280,), jnp.float32),
          pltpu.VMEM((640,), jnp.float32),
          pltpu.VMEM((1280,), jnp.float32),
      ],
  )
  def k(topic_hbm, gidx_hbm, sidx_hbm, out_hbm, parts_s, parts_c,
        tab, gv, sv, sums, cnts, h1sh, cb, tmp, ci, ob):
    c = lax.axis_index("c")
    s = lax.axis_index("s")
    n0 = s * SL
    iota = lax.iota(jnp.int32, 16)
    zeros16 = jnp.zeros((16,), jnp.float32)
    ones16 = jnp.ones((16,), jnp.float32)

    pltpu.sync_copy(topic_hbm, tab)
    pltpu.sync_copy(gidx_hbm.at[c, s], gv)
    pltpu.sync_copy(sidx_hbm.at[c, s], sv)

    def zero_buf(buf, nvregs):
      def zb(i, carry):
        buf[pl.ds(i * 16, 16)] = zeros16
        return carry
      lax.fori_loop(0, nvregs, zb, 0)

    def accum_round(with_counts):
      def chunk(j, carry):
        for i in range(8):
          g16 = gv[j, pl.ds(i * 16, 16)]
          d16 = sv[j, pl.ds(i * 16, 16)]
          m0 = plsc.load_gather(tab, [g16 * 2])
          m1 = plsc.load_gather(tab, [g16 * 2 + 1])
          d2 = d16 * 2
          plsc.addupdate_scatter(sums, [d2], m0)
          plsc.addupdate_scatter(sums, [d2 + 1], m1)
          if with_counts:
            plsc.addupdate_scatter(cnts, [d16], ones16)
        return carry
      lax.fori_loop(0, NCH, chunk, 0)

    def vadd_into(dst, nvregs):
      def body(i, carry):
        sl = pl.ds(i * 16, 16)
        dst[sl] = dst[sl] + tmp[sl]
        return carry
      lax.fori_loop(0, nvregs, body, 0)

    def combine(first_round):
      pltpu.sync_copy(sums, parts_s.at[c, s])
      if first_round:
        pltpu.sync_copy(cnts, parts_c.at[c, s])
      plsc.subcore_barrier()
      zero_buf(cb, 80)
      for t in range(NS):
        pltpu.sync_copy(parts_s.at[c, t, pl.ds(2 * n0, 2 * SL)],
                        tmp.at[pl.ds(0, 2 * SL)])
        vadd_into(cb, 79)
      if first_round:
        zero_buf(ci, 40)
        for t in range(NS):
          pltpu.sync_copy(parts_c.at[c, t, pl.ds(n0, SL)],
                          tmp.at[pl.ds(0, SL)])
          def addci(i, carry):
            sl = pl.ds(i * 16, 16)
            ci[sl] = ci[sl] + tmp[sl]
            return carry
          lax.fori_loop(0, 40, addci, 0)
      def nodes(k_, carry):
        ln = k_ * 16 + iota
        s0 = plsc.load_gather(cb, [ln * 2])
        s1 = plsc.load_gather(cb, [ln * 2 + 1])
        cn = ci[pl.ds(k_ * 16, 16)]
        inv = 1.0 / jnp.maximum(cn, 1.0)
        plsc.store_scatter(ob, [ln * 2], s0 * inv)
        plsc.store_scatter(ob, [ln * 2 + 1], s1 * inv)
        return carry
      lax.fori_loop(0, 40, nodes, 0)

    zero_buf(sums, 2 * NPAD // 16)
    zero_buf(cnts, NPAD // 16)
    accum_round(True)
    combine(True)
    pltpu.sync_copy(ob.at[pl.ds(0, 2 * SL)],
                    out_hbm.at[pl.ds(c * 2 * (2 * NPAD) + 2 * n0, 2 * SL)])
    pltpu.sync_copy(ob.at[pl.ds(0, 2 * SL)], h1sh.at[c, pl.ds(2 * n0, 2 * SL)])
    plsc.subcore_barrier()

    pltpu.sync_copy(h1sh.at[c], tab)
    zero_buf(sums, 2 * NPAD // 16)
    accum_round(False)
    combine(False)
    pltpu.sync_copy(ob.at[pl.ds(0, 2 * SL)],
                    out_hbm.at[pl.ds((c * 2 + 1) * (2 * NPAD) + 2 * n0, 2 * SL)])

  return k(topic_flat, gidx, sidx)[0]


def _edge_gather_sc(p_tab, q_tab, src, dst):
  mesh = plsc.VectorSubcoreMesh(core_axis_name="c", subcore_axis_name="s")

  @functools.partial(
      pl.kernel,
      out_type=jax.ShapeDtypeStruct((E, D), jnp.float32),
      mesh=mesh,
      compiler_params=pltpu.CompilerParams(needs_layout_passes=False, use_tc_tiling_on_sc=False),
      scratch_types=[
          pltpu.VMEM((GCH,), jnp.int32),
          pltpu.VMEM((GCH,), jnp.int32),
          pltpu.VMEM((GCH,), jnp.int32),
          pltpu.VMEM((GCH,), jnp.int32),
          pltpu.VMEM((GCH, D), jnp.float32),
          pltpu.VMEM((GCH, D), jnp.float32),
          pltpu.VMEM((GCH, D), jnp.float32),
          pltpu.VMEM((GCH, D), jnp.float32),
          pltpu.SemaphoreType.DMA,
          pltpu.SemaphoreType.DMA,
          pltpu.SemaphoreType.DMA,
          pltpu.SemaphoreType.DMA,
      ],
  )
  def k(p_hbm, q_hbm, src_hbm, dst_hbm, g_hbm,
        si_a, di_a, si_b, di_b, rp_a, rq_a, rp_b, rq_b, s1a, s2a, s1b, s2b):
    c = lax.axis_index("c")
    s = lax.axis_index("s")
    wid = s * 2 + c
    base0 = wid * EW

    def start(j, si, di, rp, rq, s1, s2):
      b = base0 + j * GCH
      pltpu.sync_copy(src_hbm.at[pl.ds(b, GCH)], si)
      pltpu.sync_copy(dst_hbm.at[pl.ds(b, GCH)], di)
      pltpu.async_copy(p_hbm.at[si], rp, s1)
      pltpu.async_copy(q_hbm.at[di], rq, s2)

    def process(j, si, di, rp, rq, s1, s2):
      pltpu.make_async_copy(p_hbm.at[si], rp, s1).wait()
      pltpu.make_async_copy(q_hbm.at[di], rq, s2).wait()

      def row(r, carry2):
        for l in range(D // 16):
          sl = pl.ds(l * 16, 16)
          rp[r, sl] = rp[r, sl] + rq[r, sl]
        return carry2
      lax.fori_loop(0, GCH, row, 0)
      pltpu.sync_copy(rp, g_hbm.at[pl.ds(base0 + j * GCH, GCH)])

    start(0, si_a, di_a, rp_a, rq_a, s1a, s2a)

    def pair(k2, carry):
      j0 = 2 * k2
      start(j0 + 1, si_b, di_b, rp_b, rq_b, s1b, s2b)
      process(j0, si_a, di_a, rp_a, rq_a, s1a, s2a)

      @pl.when(k2 < GNC // 2 - 1)
      def _():
        start(j0 + 2, si_a, di_a, rp_a, rq_a, s1a, s2a)
      process(j0 + 1, si_b, di_b, rp_b, rq_b, s1b, s2b)
      return carry
    lax.fori_loop(0, GNC // 2, pair, 0)

  return k(p_tab, q_tab, src, dst)


def _node_mm_kernel(x_ref, e_ref, nte_ref, wsx_ref, wse_ref, wdx_ref, wde_ref,
                    p_ref, q_ref):
  xb = x_ref[...]
  mask = jnp.all(xb == 0.0, axis=1, keepdims=True)
  xm = jnp.where(mask, nte_ref[...], xb)
  ex = e_ref[...]
  p_ref[...] = (jnp.dot(xm, wsx_ref[...], preferred_element_type=jnp.float32)
                + jnp.dot(ex, wse_ref[...], preferred_element_type=jnp.float32))
  q_ref[...] = (jnp.dot(xm, wdx_ref[...], preferred_element_type=jnp.float32)
                + jnp.dot(ex, wde_ref[...], preferred_element_type=jnp.float32))


def _node_mm_tc(x, extras16, nte, wsx, wse, wdx, wde):
  bn = 1000
  grid = (N // bn,)
  return pl.pallas_call(
      _node_mm_kernel,
      grid=grid,
      in_specs=[
          pl.BlockSpec((bn, D), lambda i: (i, 0)),
          pl.BlockSpec((bn, 16), lambda i: (i, 0)),
          pl.BlockSpec((1, D), lambda i: (0, 0)),
          pl.BlockSpec((D, D), lambda i: (0, 0)),
          pl.BlockSpec((16, D), lambda i: (0, 0)),
          pl.BlockSpec((D, D), lambda i: (0, 0)),
          pl.BlockSpec((16, D), lambda i: (0, 0)),
      ],
      out_specs=[
          pl.BlockSpec((bn, D), lambda i: (i, 0)),
          pl.BlockSpec((bn, D), lambda i: (i, 0)),
      ],
      out_shape=[
          jax.ShapeDtypeStruct((N, D), jnp.float32),
          jax.ShapeDtypeStruct((N, D), jnp.float32),
      ],
  )(x, extras16, nte, wsx, wse, wdx, wde)


def _edge_mlp_kernel(q_ref, ea_ref, g_ref, w1q_ref, w1e_ref, b1_ref, w2_ref,
                     b2_ref, out_ref):
  qb = q_ref[...].astype(jnp.bfloat16)
  eb = ea_ref[...].astype(jnp.bfloat16)
  h = jnp.dot(qb, w1q_ref[...], preferred_element_type=jnp.float32)
  h = h + jnp.dot(eb, w1e_ref[...], preferred_element_type=jnp.float32)
  h = h + g_ref[...] + b1_ref[...]
  h = jnp.maximum(h, 0.0)
  out_ref[...] = jnp.dot(h, w2_ref[...], preferred_element_type=jnp.float32) + b2_ref[0]


def _edge_mlp_tc(q_emb, edge_attr, g, w1q, w1ea, b1r, w2, b2):
  be = 512
  grid = (E // be,)
  return pl.pallas_call(
      _edge_mlp_kernel,
      grid=grid,
      in_specs=[
          pl.BlockSpec((be, D), lambda i: (i, 0)),
          pl.BlockSpec((be, D), lambda i: (i, 0)),
          pl.BlockSpec((be, D), lambda i: (i, 0)),
          pl.BlockSpec((D, D), lambda i: (0, 0)),
          pl.BlockSpec((D, D), lambda i: (0, 0)),
          pl.BlockSpec((1, D), lambda i: (0, 0)),
          pl.BlockSpec((D, 1), lambda i: (0, 0)),
          pl.BlockSpec(memory_space=pltpu.SMEM),
      ],
      out_specs=pl.BlockSpec((be, 1), lambda i: (i, 0)),
      out_shape=jax.ShapeDtypeStruct((E, 1), jnp.float32),
  )(q_emb, edge_attr, g, w1q, w1ea, b1r, w2, b2)


def kernel(x, edge_index, edge_attr, topic_signal, q_emb, non_text_emb,
           W1, b1, W2, b2):
  src = edge_index[0]
  dst = edge_index[1]

  pad = jnp.full((EPAD - E,), DUMMY, jnp.int32)
  srcp = jnp.concatenate([src, pad]).reshape(NS, NCH, 128)
  dstp = jnp.concatenate([dst, pad]).reshape(NS, NCH, 128)
  gidx = jnp.stack([srcp, dstp])
  sidx = jnp.stack([dstp, srcp])
  topic_flat = jnp.pad(topic_signal.reshape(-1), (0, 2 * NPAD - 2 * N))
  pe = _pe_conv_sc(topic_flat, gidx, sidx).reshape(4, 2 * NPAD)
  f1 = pe[0, :2 * N].reshape(N, 2)
  f2 = pe[1, :2 * N].reshape(N, 2)
  r1 = pe[2, :2 * N].reshape(N, 2)
  r2 = pe[3, :2 * N].reshape(N, 2)

  extras16 = jnp.concatenate(
      [topic_signal, f1, f2, r1, r2, jnp.zeros((N, 6), jnp.float32)], axis=1)
  zpad6 = jnp.zeros((6, D), jnp.float32)
  wsx = W1[128:256]
  wse = jnp.concatenate([W1[256:266], zpad6], axis=0)
  wdx = W1[394:522]
  wde = jnp.concatenate([W1[522:532], zpad6], axis=0)
  p_tab, q_tab = _node_mm_tc(x, extras16, non_text_emb, wsx, wse, wdx, wde)

  g = _edge_gather_sc(p_tab, q_tab, src, dst)

  out = _edge_mlp_tc(q_emb, edge_attr, g,
                     W1[0:128].astype(jnp.bfloat16),
                     W1[266:394].astype(jnp.bfloat16),
                     b1.reshape(1, D), W2, b2)
  return out[:, 0]


# speedup vs baseline: 9.2848x; 1.1347x over previous
import functools

import jax
import jax.numpy as jnp
from jax import lax
from jax.experimental import pallas as pl
from jax.experimental.pallas import tpu as pltpu
from jax.experimental.pallas import tpu_sc as plsc

N = 10000
E = 320000
D = 128

NS = 16
SL = 632
NPAD = NS * SL
DUMMY = NPAD - 8
EPT = E // NS
NCH = (EPT + 127) // 128
EPAD = NS * NCH * 128

EW = E // 32
GCH = 200
GNC = EW // GCH


def _pe_conv_sc(topic_flat, gidx, sidx):
  mesh = plsc.VectorSubcoreMesh(core_axis_name="c", subcore_axis_name="s")

  @functools.partial(
      pl.kernel,
      out_type=[
          jax.ShapeDtypeStruct((4 * 2 * NPAD,), jnp.float32),
          jax.ShapeDtypeStruct((2, NS, 2 * NPAD), jnp.float32),
          jax.ShapeDtypeStruct((2, NS, NPAD), jnp.float32),
      ],
      mesh=mesh,
      compiler_params=pltpu.CompilerParams(needs_layout_passes=False, use_tc_tiling_on_sc=False),
      scratch_types=[
          pltpu.VMEM((2 * NPAD,), jnp.float32),
          pltpu.VMEM((NCH, 128), jnp.int32),
          pltpu.VMEM((NCH, 128), jnp.int32),
          pltpu.VMEM((2 * NPAD,), jnp.float32),
          pltpu.VMEM((NPAD,), jnp.float32),
          pltpu.VMEM_SHARED((2, 2 * NPAD), jnp.float32),
          pltpu.VMEM((1280,), jnp.float32),
          pltpu.VMEM((1280,), jnp.float32),
          pltpu.VMEM((640,), jnp.float32),
          pltpu.VMEM((1280,), jnp.float32),
      ],
  )
  def k(topic_hbm, gidx_hbm, sidx_hbm, out_hbm, parts_s, parts_c,
        tab, gv, sv, sums, cnts, h1sh, cb, tmp, ci, ob):
    c = lax.axis_index("c")
    s = lax.axis_index("s")
    n0 = s * SL
    iota = lax.iota(jnp.int32, 16)
    zeros16 = jnp.zeros((16,), jnp.float32)
    ones16 = jnp.ones((16,), jnp.float32)

    pltpu.sync_copy(topic_hbm, tab)
    pltpu.sync_copy(gidx_hbm.at[c, s], gv)
    pltpu.sync_copy(sidx_hbm.at[c, s], sv)

    def zero_buf(buf, nvregs):
      def zb(i, carry):
        buf[pl.ds(i * 16, 16)] = zeros16
        return carry
      lax.fori_loop(0, nvregs, zb, 0)

    def accum_round(with_counts):
      def chunk(j, carry):
        for i in range(8):
          g16 = gv[j, pl.ds(i * 16, 16)]
          d16 = sv[j, pl.ds(i * 16, 16)]
          m0 = plsc.load_gather(tab, [g16 * 2])
          m1 = plsc.load_gather(tab, [g16 * 2 + 1])
          d2 = d16 * 2
          plsc.addupdate_scatter(sums, [d2], m0)
          plsc.addupdate_scatter(sums, [d2 + 1], m1)
          if with_counts:
            plsc.addupdate_scatter(cnts, [d16], ones16)
        return carry
      lax.fori_loop(0, NCH, chunk, 0)

    def vadd_into(dst, nvregs):
      def body(i, carry):
        sl = pl.ds(i * 16, 16)
        dst[sl] = dst[sl] + tmp[sl]
        return carry
      lax.fori_loop(0, nvregs, body, 0)

    def combine(first_round):
      pltpu.sync_copy(sums, parts_s.at[c, s])
      if first_round:
        pltpu.sync_copy(cnts, parts_c.at[c, s])
      plsc.subcore_barrier()
      zero_buf(cb, 80)
      for t in range(NS):
        pltpu.sync_copy(parts_s.at[c, t, pl.ds(2 * n0, 2 * SL)],
                        tmp.at[pl.ds(0, 2 * SL)])
        vadd_into(cb, 79)
      if first_round:
        zero_buf(ci, 40)
        for t in range(NS):
          pltpu.sync_copy(parts_c.at[c, t, pl.ds(n0, SL)],
                          tmp.at[pl.ds(0, SL)])
          def addci(i, carry):
            sl = pl.ds(i * 16, 16)
            ci[sl] = ci[sl] + tmp[sl]
            return carry
          lax.fori_loop(0, 40, addci, 0)
      def nodes(k_, carry):
        ln = k_ * 16 + iota
        s0 = plsc.load_gather(cb, [ln * 2])
        s1 = plsc.load_gather(cb, [ln * 2 + 1])
        cn = ci[pl.ds(k_ * 16, 16)]
        inv = 1.0 / jnp.maximum(cn, 1.0)
        plsc.store_scatter(ob, [ln * 2], s0 * inv)
        plsc.store_scatter(ob, [ln * 2 + 1], s1 * inv)
        return carry
      lax.fori_loop(0, 40, nodes, 0)

    zero_buf(sums, 2 * NPAD // 16)
    zero_buf(cnts, NPAD // 16)
    accum_round(True)
    combine(True)
    pltpu.sync_copy(ob.at[pl.ds(0, 2 * SL)],
                    out_hbm.at[pl.ds(c * 2 * (2 * NPAD) + 2 * n0, 2 * SL)])
    pltpu.sync_copy(ob.at[pl.ds(0, 2 * SL)], h1sh.at[c, pl.ds(2 * n0, 2 * SL)])
    plsc.subcore_barrier()

    pltpu.sync_copy(h1sh.at[c], tab)
    zero_buf(sums, 2 * NPAD // 16)
    accum_round(False)
    combine(False)
    pltpu.sync_copy(ob.at[pl.ds(0, 2 * SL)],
                    out_hbm.at[pl.ds((c * 2 + 1) * (2 * NPAD) + 2 * n0, 2 * SL)])

  return k(topic_flat, gidx, sidx)[0]


def _edge_gather_sc(p_tab, q_tab, src, dst, slice_k, nsl):
  mesh = plsc.VectorSubcoreMesh(core_axis_name="c", subcore_axis_name="s")
  esl = E // nsl
  ewk = esl // 32
  nchk = ewk // GCH

  @functools.partial(
      pl.kernel,
      out_type=jax.ShapeDtypeStruct((esl, D), jnp.float32),
      mesh=mesh,
      compiler_params=pltpu.CompilerParams(needs_layout_passes=False, use_tc_tiling_on_sc=False),
      scratch_types=[
          pltpu.VMEM((GCH,), jnp.int32),
          pltpu.VMEM((GCH,), jnp.int32),
          pltpu.VMEM((GCH,), jnp.int32),
          pltpu.VMEM((GCH,), jnp.int32),
          pltpu.VMEM((GCH, D), jnp.float32),
          pltpu.VMEM((GCH, D), jnp.float32),
          pltpu.VMEM((GCH, D), jnp.float32),
          pltpu.VMEM((GCH, D), jnp.float32),
          pltpu.SemaphoreType.DMA,
          pltpu.SemaphoreType.DMA,
          pltpu.SemaphoreType.DMA,
          pltpu.SemaphoreType.DMA,
      ],
  )
  def k(p_hbm, q_hbm, src_hbm, dst_hbm, g_hbm,
        si_a, di_a, si_b, di_b, rp_a, rq_a, rp_b, rq_b, s1a, s2a, s1b, s2b):
    c = lax.axis_index("c")
    s = lax.axis_index("s")
    wid = s * 2 + c
    ebase = slice_k * esl + wid * ewk
    obase = wid * ewk

    def start(j, si, di, rp, rq, s1, s2):
      b = ebase + j * GCH
      pltpu.sync_copy(src_hbm.at[pl.ds(b, GCH)], si)
      pltpu.sync_copy(dst_hbm.at[pl.ds(b, GCH)], di)
      pltpu.async_copy(p_hbm.at[si], rp, s1)
      pltpu.async_copy(q_hbm.at[di], rq, s2)

    def process(j, si, di, rp, rq, s1, s2):
      pltpu.make_async_copy(p_hbm.at[si], rp, s1).wait()
      pltpu.make_async_copy(q_hbm.at[di], rq, s2).wait()

      def row(r, carry2):
        for l in range(D // 16):
          sl = pl.ds(l * 16, 16)
          rp[r, sl] = rp[r, sl] + rq[r, sl]
        return carry2
      lax.fori_loop(0, GCH, row, 0)
      pltpu.sync_copy(rp, g_hbm.at[pl.ds(obase + j * GCH, GCH)])

    start(0, si_a, di_a, rp_a, rq_a, s1a, s2a)

    def pair(k2, carry):
      j0 = 2 * k2
      start(j0 + 1, si_b, di_b, rp_b, rq_b, s1b, s2b)
      process(j0, si_a, di_a, rp_a, rq_a, s1a, s2a)

      @pl.when(j0 + 2 < nchk)
      def _():
        start(j0 + 2, si_a, di_a, rp_a, rq_a, s1a, s2a)
      process(j0 + 1, si_b, di_b, rp_b, rq_b, s1b, s2b)
      return carry
    lax.fori_loop(0, nchk // 2, pair, 0)
    if nchk % 2:
      process(nchk - 1, si_a, di_a, rp_a, rq_a, s1a, s2a)

  return k(p_tab, q_tab, src, dst)


def _node_mm_kernel(x_ref, e_ref, nte_ref, wsx_ref, wse_ref, wdx_ref, wde_ref,
                    p_ref, q_ref):
  xb = x_ref[...]
  mask = jnp.all(xb == 0.0, axis=1, keepdims=True)
  xm = jnp.where(mask, nte_ref[...], xb)
  ex = e_ref[...]
  p_ref[...] = (jnp.dot(xm, wsx_ref[...], preferred_element_type=jnp.float32)
                + jnp.dot(ex, wse_ref[...], preferred_element_type=jnp.float32))
  q_ref[...] = (jnp.dot(xm, wdx_ref[...], preferred_element_type=jnp.float32)
                + jnp.dot(ex, wde_ref[...], preferred_element_type=jnp.float32))


def _node_mm_tc(x, extras16, nte, wsx, wse, wdx, wde):
  bn = 1000
  grid = (N // bn,)
  return pl.pallas_call(
      _node_mm_kernel,
      grid=grid,
      in_specs=[
          pl.BlockSpec((bn, D), lambda i: (i, 0)),
          pl.BlockSpec((bn, 16), lambda i: (i, 0)),
          pl.BlockSpec((1, D), lambda i: (0, 0)),
          pl.BlockSpec((D, D), lambda i: (0, 0)),
          pl.BlockSpec((16, D), lambda i: (0, 0)),
          pl.BlockSpec((D, D), lambda i: (0, 0)),
          pl.BlockSpec((16, D), lambda i: (0, 0)),
      ],
      out_specs=[
          pl.BlockSpec((bn, D), lambda i: (i, 0)),
          pl.BlockSpec((bn, D), lambda i: (i, 0)),
      ],
      out_shape=[
          jax.ShapeDtypeStruct((N, D), jnp.float32),
          jax.ShapeDtypeStruct((N, D), jnp.float32),
      ],
  )(x, extras16, nte, wsx, wse, wdx, wde)


def _edge_mlp_kernel(q_ref, ea_ref, g_ref, w1q_ref, w1e_ref, b1_ref, w2_ref,
                     b2_ref, out_ref):
  qb = q_ref[...].astype(jnp.bfloat16)
  eb = ea_ref[...].astype(jnp.bfloat16)
  h = jnp.dot(qb, w1q_ref[...], preferred_element_type=jnp.float32)
  h = h + jnp.dot(eb, w1e_ref[...], preferred_element_type=jnp.float32)
  h = h + g_ref[...] + b1_ref[...]
  h = jnp.maximum(h, 0.0)
  out_ref[...] = jnp.dot(h, w2_ref[...], preferred_element_type=jnp.float32) + b2_ref[0]


def _edge_mlp_tc(q_emb, edge_attr, g, w1q, w1ea, b1r, w2, b2, slice_k, nsl):
  be = 640
  esl = E // nsl
  off = slice_k * (esl // be)
  grid = (esl // be,)
  return pl.pallas_call(
      _edge_mlp_kernel,
      grid=grid,
      in_specs=[
          pl.BlockSpec((be, D), lambda i: (i + off, 0)),
          pl.BlockSpec((be, D), lambda i: (i + off, 0)),
          pl.BlockSpec((be, D), lambda i: (i, 0)),
          pl.BlockSpec((D, D), lambda i: (0, 0)),
          pl.BlockSpec((D, D), lambda i: (0, 0)),
          pl.BlockSpec((1, D), lambda i: (0, 0)),
          pl.BlockSpec((D, 1), lambda i: (0, 0)),
          pl.BlockSpec(memory_space=pltpu.SMEM),
      ],
      out_specs=pl.BlockSpec((be, 1), lambda i: (i, 0)),
      out_shape=jax.ShapeDtypeStruct((esl, 1), jnp.float32),
  )(q_emb, edge_attr, g, w1q, w1ea, b1r, w2, b2)


def kernel(x, edge_index, edge_attr, topic_signal, q_emb, non_text_emb,
           W1, b1, W2, b2):
  src = edge_index[0]
  dst = edge_index[1]

  pad = jnp.full((EPAD - E,), DUMMY, jnp.int32)
  srcp = jnp.concatenate([src, pad]).reshape(NS, NCH, 128)
  dstp = jnp.concatenate([dst, pad]).reshape(NS, NCH, 128)
  gidx = jnp.stack([srcp, dstp])
  sidx = jnp.stack([dstp, srcp])
  topic_flat = jnp.pad(topic_signal.reshape(-1), (0, 2 * NPAD - 2 * N))
  pe = _pe_conv_sc(topic_flat, gidx, sidx).reshape(4, 2 * NPAD)
  f1 = pe[0, :2 * N].reshape(N, 2)
  f2 = pe[1, :2 * N].reshape(N, 2)
  r1 = pe[2, :2 * N].reshape(N, 2)
  r2 = pe[3, :2 * N].reshape(N, 2)

  extras16 = jnp.concatenate(
      [topic_signal, f1, f2, r1, r2, jnp.zeros((N, 6), jnp.float32)], axis=1)
  zpad6 = jnp.zeros((6, D), jnp.float32)
  wsx = W1[128:256]
  wse = jnp.concatenate([W1[256:266], zpad6], axis=0)
  wdx = W1[394:522]
  wde = jnp.concatenate([W1[522:532], zpad6], axis=0)
  p_tab, q_tab = _node_mm_tc(x, extras16, non_text_emb, wsx, wse, wdx, wde)

  nsl = 2
  w1q = W1[0:128].astype(jnp.bfloat16)
  w1ea = W1[266:394].astype(jnp.bfloat16)
  b1r = b1.reshape(1, D)
  outs = []
  for sk in range(nsl):
    g = _edge_gather_sc(p_tab, q_tab, src, dst, sk, nsl)
    outs.append(_edge_mlp_tc(q_emb, edge_attr, g, w1q, w1ea, b1r, W2, b2,
                             sk, nsl)[:, 0])
  return jnp.concatenate(outs)


# speedup vs baseline: 11.4030x; 1.2281x over previous
import functools

import jax
import jax.numpy as jnp
from jax import lax
from jax.experimental import pallas as pl
from jax.experimental.pallas import tpu as pltpu
from jax.experimental.pallas import tpu_sc as plsc

N = 10000
E = 320000
D = 128

NS = 16
SL = 632
NPAD = NS * SL
DUMMY = NPAD - 8
EPT = E // NS
NCH = (EPT + 127) // 128
EPAD = NS * NCH * 128

EW = E // 32
GCH = 200
GNC = EW // GCH


def _pe_conv_sc(topic_flat, gidx, sidx):
  mesh = plsc.VectorSubcoreMesh(core_axis_name="c", subcore_axis_name="s")

  @functools.partial(
      pl.kernel,
      out_type=[
          jax.ShapeDtypeStruct((4 * 2 * NPAD,), jnp.float32),
          jax.ShapeDtypeStruct((2, NS, 2 * NPAD), jnp.float32),
          jax.ShapeDtypeStruct((2, NS, NPAD), jnp.float32),
      ],
      mesh=mesh,
      compiler_params=pltpu.CompilerParams(needs_layout_passes=False, use_tc_tiling_on_sc=False),
      scratch_types=[
          pltpu.VMEM((2 * NPAD,), jnp.float32),
          pltpu.VMEM((NCH, 128), jnp.int32),
          pltpu.VMEM((NCH, 128), jnp.int32),
          pltpu.VMEM((2 * NPAD,), jnp.float32),
          pltpu.VMEM((NPAD,), jnp.float32),
          pltpu.VMEM_SHARED((2, 2 * NPAD), jnp.float32),
          pltpu.VMEM((1280,), jnp.float32),
          pltpu.VMEM((1280,), jnp.float32),
          pltpu.VMEM((640,), jnp.float32),
          pltpu.VMEM((1280,), jnp.float32),
      ],
  )
  def k(topic_hbm, gidx_hbm, sidx_hbm, out_hbm, parts_s, parts_c,
        tab, gv, sv, sums, cnts, h1sh, cb, tmp, ci, ob):
    c = lax.axis_index("c")
    s = lax.axis_index("s")
    n0 = s * SL
    iota = lax.iota(jnp.int32, 16)
    zeros16 = jnp.zeros((16,), jnp.float32)
    ones16 = jnp.ones((16,), jnp.float32)

    pltpu.sync_copy(topic_hbm, tab)
    pltpu.sync_copy(gidx_hbm.at[c, s], gv)
    pltpu.sync_copy(sidx_hbm.at[c, s], sv)

    def zero_buf(buf, nvregs):
      def zb(i, carry):
        buf[pl.ds(i * 16, 16)] = zeros16
        return carry
      lax.fori_loop(0, nvregs, zb, 0)

    def accum_round(with_counts):
      def chunk(j, carry):
        for i in range(8):
          g16 = gv[j, pl.ds(i * 16, 16)]
          d16 = sv[j, pl.ds(i * 16, 16)]
          m0 = plsc.load_gather(tab, [g16 * 2])
          m1 = plsc.load_gather(tab, [g16 * 2 + 1])
          d2 = d16 * 2
          plsc.addupdate_scatter(sums, [d2], m0)
          plsc.addupdate_scatter(sums, [d2 + 1], m1)
          if with_counts:
            plsc.addupdate_scatter(cnts, [d16], ones16)
        return carry
      lax.fori_loop(0, NCH, chunk, 0)

    def vadd_into(dst, nvregs):
      def body(i, carry):
        sl = pl.ds(i * 16, 16)
        dst[sl] = dst[sl] + tmp[sl]
        return carry
      lax.fori_loop(0, nvregs, body, 0)

    def combine(first_round):
      pltpu.sync_copy(sums, parts_s.at[c, s])
      if first_round:
        pltpu.sync_copy(cnts, parts_c.at[c, s])
      plsc.subcore_barrier()
      zero_buf(cb, 80)
      for t in range(NS):
        pltpu.sync_copy(parts_s.at[c, t, pl.ds(2 * n0, 2 * SL)],
                        tmp.at[pl.ds(0, 2 * SL)])
        vadd_into(cb, 79)
      if first_round:
        zero_buf(ci, 40)
        for t in range(NS):
          pltpu.sync_copy(parts_c.at[c, t, pl.ds(n0, SL)],
                          tmp.at[pl.ds(0, SL)])
          def addci(i, carry):
            sl = pl.ds(i * 16, 16)
            ci[sl] = ci[sl] + tmp[sl]
            return carry
          lax.fori_loop(0, 40, addci, 0)
      def nodes(k_, carry):
        ln = k_ * 16 + iota
        s0 = plsc.load_gather(cb, [ln * 2])
        s1 = plsc.load_gather(cb, [ln * 2 + 1])
        cn = ci[pl.ds(k_ * 16, 16)]
        inv = 1.0 / jnp.maximum(cn, 1.0)
        plsc.store_scatter(ob, [ln * 2], s0 * inv)
        plsc.store_scatter(ob, [ln * 2 + 1], s1 * inv)
        return carry
      lax.fori_loop(0, 40, nodes, 0)

    zero_buf(sums, 2 * NPAD // 16)
    zero_buf(cnts, NPAD // 16)
    accum_round(True)
    combine(True)
    pltpu.sync_copy(ob.at[pl.ds(0, 2 * SL)],
                    out_hbm.at[pl.ds(c * 2 * (2 * NPAD) + 2 * n0, 2 * SL)])
    pltpu.sync_copy(ob.at[pl.ds(0, 2 * SL)], h1sh.at[c, pl.ds(2 * n0, 2 * SL)])
    plsc.subcore_barrier()

    pltpu.sync_copy(h1sh.at[c], tab)
    zero_buf(sums, 2 * NPAD // 16)
    accum_round(False)
    combine(False)
    pltpu.sync_copy(ob.at[pl.ds(0, 2 * SL)],
                    out_hbm.at[pl.ds((c * 2 + 1) * (2 * NPAD) + 2 * n0, 2 * SL)])

  return k(topic_flat, gidx, sidx)[0]


def _edge_gather_sc(p_tab, q_tab, src, dst, slice_k, nsl):
  mesh = plsc.VectorSubcoreMesh(core_axis_name="c", subcore_axis_name="s")
  esl = E // nsl
  ewk = esl // 32
  nchk = ewk // GCH

  @functools.partial(
      pl.kernel,
      out_type=jax.ShapeDtypeStruct((esl, D), jnp.float32),
      mesh=mesh,
      compiler_params=pltpu.CompilerParams(needs_layout_passes=False, use_tc_tiling_on_sc=False),
      scratch_types=[
          pltpu.VMEM((GCH,), jnp.int32),
          pltpu.VMEM((GCH,), jnp.int32),
          pltpu.VMEM((GCH,), jnp.int32),
          pltpu.VMEM((GCH,), jnp.int32),
          pltpu.VMEM((GCH, D), jnp.float32),
          pltpu.VMEM((GCH, D), jnp.float32),
          pltpu.VMEM((GCH, D), jnp.float32),
          pltpu.VMEM((GCH, D), jnp.float32),
          pltpu.SemaphoreType.DMA,
          pltpu.SemaphoreType.DMA,
          pltpu.SemaphoreType.DMA,
          pltpu.SemaphoreType.DMA,
      ],
  )
  def k(p_hbm, q_hbm, src_hbm, dst_hbm, g_hbm,
        si_a, di_a, si_b, di_b, rp_a, rq_a, rp_b, rq_b, s1a, s2a, s1b, s2b):
    c = lax.axis_index("c")
    s = lax.axis_index("s")
    wid = s * 2 + c
    ebase = slice_k * esl + wid * ewk
    obase = wid * ewk

    def start(j, si, di, rp, rq, s1, s2):
      b = ebase + j * GCH
      pltpu.sync_copy(src_hbm.at[pl.ds(b, GCH)], si)
      pltpu.sync_copy(dst_hbm.at[pl.ds(b, GCH)], di)
      pltpu.async_copy(p_hbm.at[si], rp, s1)
      pltpu.async_copy(q_hbm.at[di], rq, s2)

    def process(j, si, di, rp, rq, s1, s2):
      pltpu.make_async_copy(p_hbm.at[si], rp, s1).wait()
      pltpu.make_async_copy(q_hbm.at[di], rq, s2).wait()

      def row(r, carry2):
        for l in range(D // 16):
          sl = pl.ds(l * 16, 16)
          rp[r, sl] = rp[r, sl] + rq[r, sl]
        return carry2
      lax.fori_loop(0, GCH, row, 0)
      pltpu.sync_copy(rp, g_hbm.at[pl.ds(obase + j * GCH, GCH)])

    start(0, si_a, di_a, rp_a, rq_a, s1a, s2a)

    def pair(k2, carry):
      j0 = 2 * k2
      start(j0 + 1, si_b, di_b, rp_b, rq_b, s1b, s2b)
      process(j0, si_a, di_a, rp_a, rq_a, s1a, s2a)

      @pl.when(j0 + 2 < nchk)
      def _():
        start(j0 + 2, si_a, di_a, rp_a, rq_a, s1a, s2a)
      process(j0 + 1, si_b, di_b, rp_b, rq_b, s1b, s2b)
      return carry
    lax.fori_loop(0, nchk // 2, pair, 0)
    if nchk % 2:
      process(nchk - 1, si_a, di_a, rp_a, rq_a, s1a, s2a)

  return k(p_tab, q_tab, src, dst)


def _node_mm_kernel(x_ref, e_ref, nte_ref, wsx_ref, wse_ref, wdx_ref, wde_ref,
                    p_ref, q_ref):
  xb = x_ref[...]
  mask = jnp.all(xb == 0.0, axis=1, keepdims=True)
  xm = jnp.where(mask, nte_ref[...], xb)
  ex = e_ref[...]
  p_ref[...] = (jnp.dot(xm, wsx_ref[...], preferred_element_type=jnp.float32)
                + jnp.dot(ex, wse_ref[...], preferred_element_type=jnp.float32))
  q_ref[...] = (jnp.dot(xm, wdx_ref[...], preferred_element_type=jnp.float32)
                + jnp.dot(ex, wde_ref[...], preferred_element_type=jnp.float32))


def _node_mm_tc(x, extras16, nte, wsx, wse, wdx, wde):
  bn = 1000
  grid = (N // bn,)
  return pl.pallas_call(
      _node_mm_kernel,
      grid=grid,
      in_specs=[
          pl.BlockSpec((bn, D), lambda i: (i, 0)),
          pl.BlockSpec((bn, 16), lambda i: (i, 0)),
          pl.BlockSpec((1, D), lambda i: (0, 0)),
          pl.BlockSpec((D, D), lambda i: (0, 0)),
          pl.BlockSpec((16, D), lambda i: (0, 0)),
          pl.BlockSpec((D, D), lambda i: (0, 0)),
          pl.BlockSpec((16, D), lambda i: (0, 0)),
      ],
      out_specs=[
          pl.BlockSpec((bn, D), lambda i: (i, 0)),
          pl.BlockSpec((bn, D), lambda i: (i, 0)),
      ],
      out_shape=[
          jax.ShapeDtypeStruct((N, D), jnp.float32),
          jax.ShapeDtypeStruct((N, D), jnp.float32),
      ],
  )(x, extras16, nte, wsx, wse, wdx, wde)


def _edge_mlp_kernel(q_ref, ea_ref, g_ref, w1q_ref, w1e_ref, b1_ref, w2_ref,
                     b2_ref, out_ref):
  qb = q_ref[...].astype(jnp.bfloat16)
  eb = ea_ref[...].astype(jnp.bfloat16)
  h = jnp.dot(qb, w1q_ref[...], preferred_element_type=jnp.float32)
  h = h + jnp.dot(eb, w1e_ref[...], preferred_element_type=jnp.float32)
  h = h + g_ref[...] + b1_ref[...]
  h = jnp.maximum(h, 0.0)
  out_ref[...] = jnp.dot(h, w2_ref[...], preferred_element_type=jnp.float32) + b2_ref[0]


def _edge_mlp_tc(q_emb, edge_attr, g, w1q, w1ea, b1r, w2, b2, slice_k, nsl):
  be = 1600
  esl = E // nsl
  off = slice_k * (esl // be)
  grid = (esl // be,)
  return pl.pallas_call(
      _edge_mlp_kernel,
      grid=grid,
      in_specs=[
          pl.BlockSpec((be, D), lambda i: (i + off, 0)),
          pl.BlockSpec((be, D), lambda i: (i + off, 0)),
          pl.BlockSpec((be, D), lambda i: (i, 0)),
          pl.BlockSpec((D, D), lambda i: (0, 0)),
          pl.BlockSpec((D, D), lambda i: (0, 0)),
          pl.BlockSpec((1, D), lambda i: (0, 0)),
          pl.BlockSpec((D, 1), lambda i: (0, 0)),
          pl.BlockSpec(memory_space=pltpu.SMEM),
      ],
      out_specs=pl.BlockSpec((be, 1), lambda i: (i, 0)),
      out_shape=jax.ShapeDtypeStruct((esl, 1), jnp.float32),
  )(q_emb, edge_attr, g, w1q, w1ea, b1r, w2, b2)


def kernel(x, edge_index, edge_attr, topic_signal, q_emb, non_text_emb,
           W1, b1, W2, b2):
  src = edge_index[0]
  dst = edge_index[1]

  pad = jnp.full((EPAD - E,), DUMMY, jnp.int32)
  srcp = jnp.concatenate([src, pad]).reshape(NS, NCH, 128)
  dstp = jnp.concatenate([dst, pad]).reshape(NS, NCH, 128)
  gidx = jnp.stack([srcp, dstp])
  sidx = jnp.stack([dstp, srcp])
  topic_flat = jnp.pad(topic_signal.reshape(-1), (0, 2 * NPAD - 2 * N))
  pe = _pe_conv_sc(topic_flat, gidx, sidx).reshape(4, 2 * NPAD)
  f1 = pe[0, :2 * N].reshape(N, 2)
  f2 = pe[1, :2 * N].reshape(N, 2)
  r1 = pe[2, :2 * N].reshape(N, 2)
  r2 = pe[3, :2 * N].reshape(N, 2)

  extras16 = jnp.concatenate(
      [topic_signal, f1, f2, r1, r2, jnp.zeros((N, 6), jnp.float32)], axis=1)
  zpad6 = jnp.zeros((6, D), jnp.float32)
  wsx = W1[128:256]
  wse = jnp.concatenate([W1[256:266], zpad6], axis=0)
  wdx = W1[394:522]
  wde = jnp.concatenate([W1[522:532], zpad6], axis=0)
  p_tab, q_tab = _node_mm_tc(x, extras16, non_text_emb, wsx, wse, wdx, wde)

  nsl = 2
  w1q = W1[0:128].astype(jnp.bfloat16)
  w1ea = W1[266:394].astype(jnp.bfloat16)
  b1r = b1.reshape(1, D)
  outs = []
  for sk in range(nsl):
    g = _edge_gather_sc(p_tab, q_tab, src, dst, sk, nsl)
    outs.append(_edge_mlp_tc(q_emb, edge_attr, g, w1q, w1ea, b1r, W2, b2,
                             sk, nsl)[:, 0])
  return jnp.concatenate(outs)
